# Initial kernel scaffold; baseline (speedup 1.0000x reference)
import jax
import jax.numpy as jnp
from jax.experimental import pallas as pl


def kernel(x, node_types, W1, b1, W2, b2):
    raise NotImplementedError("write your pallas kernel here")


# speedup vs baseline: 1.7248x; 1.7248x over previous
import functools

import jax
import jax.numpy as jnp
from jax.experimental import pallas as pl
from jax.experimental.pallas import tpu as pltpu

_NUM_TYPES = 17
_IN = 128
_HID = 128
_OUT = 64


def _mlp_body(t_ref, x_ref, w1_ref, b1_ref, w2_ref, b2_ref, o_ref):
    r = x_ref.shape[0]
    x = x_ref[...]
    t = t_ref[...]
    h = jnp.dot(x, w1_ref[...], preferred_element_type=jnp.float32)
    h = jax.nn.relu(h + b1_ref[...])
    col_type = jax.lax.broadcasted_iota(jnp.int32, (r, _NUM_TYPES * _HID), 1) // _HID
    h = jnp.where(col_type == t, h, 0.0)
    o = jnp.dot(h, w2_ref[...], preferred_element_type=jnp.float32)
    onehot = (jax.lax.broadcasted_iota(jnp.int32, (r, _NUM_TYPES), 1) == t).astype(
        jnp.float32
    )
    o_ref[...] = o + jnp.dot(onehot, b2_ref[...], preferred_element_type=jnp.float32)


@functools.partial(jax.jit, static_argnames=("rows", "interpret"))
def _mlp_dense(x, types2d, w1c, b1c, w2c, b2, rows=1000, interpret=False):
    n = x.shape[0]
    grid = (n // rows,)
    return pl.pallas_call(
        _mlp_body,
        grid=grid,
        in_specs=[
            pl.BlockSpec((rows, 1), lambda b: (b, 0)),
            pl.BlockSpec((rows, _IN), lambda b: (b, 0)),
            pl.BlockSpec((_IN, _NUM_TYPES * _HID), lambda b: (0, 0)),
            pl.BlockSpec((1, _NUM_TYPES * _HID), lambda b: (0, 0)),
            pl.BlockSpec((_NUM_TYPES * _HID, _OUT), lambda b: (0, 0)),
            pl.BlockSpec((_NUM_TYPES, _OUT), lambda b: (0, 0)),
        ],
        out_specs=pl.BlockSpec((rows, _OUT), lambda b: (b, 0)),
        out_shape=jax.ShapeDtypeStruct((n, _OUT), jnp.float32),
        compiler_params=pltpu.CompilerParams(
            dimension_semantics=("arbitrary",),
        ),
        interpret=interpret,
    )(types2d, x, w1c, b1c, w2c, b2)


def kernel(x, node_types, W1, b1, W2, b2):
    n = x.shape[0]
    types2d = node_types.reshape(n, 1)
    w1c = jnp.transpose(W1, (1, 0, 2)).reshape(_IN, _NUM_TYPES * _HID)
    b1c = b1.reshape(1, _NUM_TYPES * _HID)
    w2c = W2.reshape(_NUM_TYPES * _HID, _OUT)
    return _mlp_dense(x, types2d, w1c, b1c, w2c, b2)
